# baseline (speedup 1.0000x reference)
import functools

import jax
import jax.numpy as jnp
from jax.experimental import pallas as pl
from jax.experimental.pallas import tpu as pltpu

_IN = 3
_HID = 64
_OUT = 2


def _mlp_kernel(x_ref, w1t_ref, b1_ref, w2t_ref, b2_ref, o_ref):
    xb = x_ref[...]
    h = jnp.dot(xb, w1t_ref[...], preferred_element_type=jnp.float32)
    h = jnp.maximum(h + b1_ref[...], 0.0).astype(jnp.bfloat16)
    y = jnp.dot(h, w2t_ref[...].astype(jnp.bfloat16),
                preferred_element_type=jnp.float32)
    o_ref[...] = y + b2_ref[...]


@functools.partial(jax.jit, static_argnames=("tn",))
def _forward(x, w1, b1, w2, b2, *, tn=4096):
    n = x.shape[0]
    assert n % tn == 0
    grid = (n // tn,)

    w1t = w1.T
    b1r = b1.reshape(1, _HID)
    w2t = w2.T
    b2r = b2.reshape(1, _OUT)

    out = pl.pallas_call(
        _mlp_kernel,
        out_shape=jax.ShapeDtypeStruct((n, _OUT), jnp.float32),
        grid_spec=pl.GridSpec(
            grid=grid,
            in_specs=[
                pl.BlockSpec((tn, _IN), lambda i: (i, 0)),
                pl.BlockSpec((_IN, _HID), lambda i: (0, 0)),
                pl.BlockSpec((1, _HID), lambda i: (0, 0)),
                pl.BlockSpec((_HID, _OUT), lambda i: (0, 0)),
                pl.BlockSpec((1, _OUT), lambda i: (0, 0)),
            ],
            out_specs=pl.BlockSpec((tn, _OUT), lambda i: (i, 0)),
        ),
        compiler_params=pltpu.CompilerParams(
            dimension_semantics=("parallel",),
        ),
    )(x, w1t, b1r, w2t, b2r)

    return out


def kernel(x, w1, b1, w2, b2):
    return _forward(x, w1, b1, w2, b2, tn=4096)


# speedup vs baseline: 14.0584x; 14.0584x over previous
import functools

import jax
import jax.numpy as jnp
from jax.experimental import pallas as pl
from jax.experimental.pallas import tpu as pltpu

_IN = 3
_HID = 64
_OUT = 2


def _mlp_kernel(xt_ref, w1_ref, b1_ref, w2_ref, b2_ref, o_ref):
    xt = xt_ref[...]
    h = jnp.dot(w1_ref[...], xt, preferred_element_type=jnp.float32)
    h = jnp.maximum(h + b1_ref[...], 0.0)
    y = jnp.dot(w2_ref[...], h, preferred_element_type=jnp.float32) + b2_ref[...]
    o_ref[...] = y.astype(o_ref.dtype)


@functools.partial(jax.jit, static_argnames=("tn",))
def _forward(x, w1, b1, w2, b2, *, tn=16384):
    n = x.shape[0]
    n_128 = max(128, ((n + 127) // 128) * 128)
    tile = min(tn, n_128)
    n_pad = ((n_128 + tile - 1) // tile) * tile
    grid = (n_pad // tile,)

    xt = jnp.pad(x.T, ((0, 0), (0, n_pad - n)))
    b1c = b1.reshape(_HID, 1)
    b2c = b2.reshape(_OUT, 1)

    out_t = pl.pallas_call(
        _mlp_kernel,
        out_shape=jax.ShapeDtypeStruct((_OUT, n_pad), jnp.float32),
        grid_spec=pl.GridSpec(
            grid=grid,
            in_specs=[
                pl.BlockSpec((_IN, tile), lambda i: (0, i)),
                pl.BlockSpec((_HID, _IN), lambda i: (0, 0)),
                pl.BlockSpec((_HID, 1), lambda i: (0, 0)),
                pl.BlockSpec((_OUT, _HID), lambda i: (0, 0)),
                pl.BlockSpec((_OUT, 1), lambda i: (0, 0)),
            ],
            out_specs=pl.BlockSpec((_OUT, tile), lambda i: (0, i)),
        ),
        compiler_params=pltpu.CompilerParams(
            dimension_semantics=("parallel",),
        ),
    )(xt, w1, b1c, w2, b2c)

    return out_t[:, :n].T


def kernel(x, w1, b1, w2, b2):
    return _forward(x, w1, b1, w2, b2, tn=16384)


# speedup vs baseline: 16.5984x; 1.1807x over previous
import functools

import jax
import jax.numpy as jnp
from jax.experimental import pallas as pl
from jax.experimental.pallas import tpu as pltpu

_IN = 3
_HID = 64
_OUT = 2


def _mlp_kernel(xt_ref, w1_ref, b1_ref, w2_ref, b2_ref, o_ref):
    xt = xt_ref[...]
    h = jnp.dot(w1_ref[...], xt, preferred_element_type=jnp.float32)
    h = jnp.maximum(h + b1_ref[...], 0.0)
    y = jnp.dot(w2_ref[...], h, preferred_element_type=jnp.float32) + b2_ref[...]
    o_ref[...] = y.astype(o_ref.dtype)


@functools.partial(jax.jit, static_argnames=("tn",))
def _forward(x, w1, b1, w2, b2, *, tn=65536):
    n = x.shape[0]
    n_128 = max(128, ((n + 127) // 128) * 128)
    tile = min(tn, n_128)
    n_pad = ((n_128 + tile - 1) // tile) * tile
    grid = (n_pad // tile,)

    xt = jnp.pad(x.T, ((0, 0), (0, n_pad - n)))
    b1c = b1.reshape(_HID, 1)
    b2c = b2.reshape(_OUT, 1)

    out_t = pl.pallas_call(
        _mlp_kernel,
        out_shape=jax.ShapeDtypeStruct((_OUT, n_pad), jnp.float32),
        grid_spec=pl.GridSpec(
            grid=grid,
            in_specs=[
                pl.BlockSpec((_IN, tile), lambda i: (0, i)),
                pl.BlockSpec((_HID, _IN), lambda i: (0, 0)),
                pl.BlockSpec((_HID, 1), lambda i: (0, 0)),
                pl.BlockSpec((_OUT, _HID), lambda i: (0, 0)),
                pl.BlockSpec((_OUT, 1), lambda i: (0, 0)),
            ],
            out_specs=pl.BlockSpec((_OUT, tile), lambda i: (0, i)),
        ),
        compiler_params=pltpu.CompilerParams(
            dimension_semantics=("parallel",),
        ),
    )(xt, w1, b1c, w2, b2c)

    return out_t[:, :n].T


def kernel(x, w1, b1, w2, b2):
    return _forward(x, w1, b1, w2, b2, tn=65536)


# speedup vs baseline: 17.0579x; 1.0277x over previous
import functools

import jax
import jax.numpy as jnp
from jax.experimental import pallas as pl
from jax.experimental.pallas import tpu as pltpu

_IN = 3
_HID = 64
_OUT = 2


def _mlp_kernel(xt_ref, w1_ref, b1_ref, w2_ref, b2_ref, o_ref):
    xt = xt_ref[...]
    h = jnp.dot(w1_ref[...], xt, preferred_element_type=jnp.float32)
    h = jnp.maximum(h + b1_ref[...], 0.0)
    y = jnp.dot(w2_ref[...], h, preferred_element_type=jnp.float32) + b2_ref[...]
    o_ref[...] = y.astype(o_ref.dtype)


@functools.partial(jax.jit, static_argnames=("tn",))
def _forward(x, w1, b1, w2, b2, *, tn=262144):
    n = x.shape[0]
    n_128 = max(128, ((n + 127) // 128) * 128)
    tile = min(tn, n_128)
    n_pad = ((n_128 + tile - 1) // tile) * tile
    grid = (n_pad // tile,)

    xt = jnp.pad(x.T, ((0, 0), (0, n_pad - n)))
    b1c = b1.reshape(_HID, 1)
    b2c = b2.reshape(_OUT, 1)

    out_t = pl.pallas_call(
        _mlp_kernel,
        out_shape=jax.ShapeDtypeStruct((_OUT, n_pad), jnp.float32),
        grid_spec=pl.GridSpec(
            grid=grid,
            in_specs=[
                pl.BlockSpec((_IN, tile), lambda i: (0, i)),
                pl.BlockSpec((_HID, _IN), lambda i: (0, 0)),
                pl.BlockSpec((_HID, 1), lambda i: (0, 0)),
                pl.BlockSpec((_OUT, _HID), lambda i: (0, 0)),
                pl.BlockSpec((_OUT, 1), lambda i: (0, 0)),
            ],
            out_specs=pl.BlockSpec((_OUT, tile), lambda i: (0, i)),
        ),
        compiler_params=pltpu.CompilerParams(
            dimension_semantics=("parallel",),
        ),
    )(xt, w1, b1c, w2, b2c)

    return out_t[:, :n].T


def kernel(x, w1, b1, w2, b2):
    return _forward(x, w1, b1, w2, b2, tn=262144)
